# baseline (speedup 1.0000x reference)
import functools

import jax
import jax.numpy as jnp
from jax import lax
from jax.experimental import pallas as pl
from jax.experimental.pallas import tpu as pltpu
from jax.experimental.pallas import tpu_sc as plsc

NC = 2
NS = 16
NW = NC * NS

BATCH = 4096
SEQ = 200
D_MODEL = 64
TOTAL = BATCH * SEQ
B_PER_W = TOTAL // NW
CHUNK = 512
N_CHUNKS = B_PER_W // CHUNK


def _gather_body(idx_hbm, table_hbm, out_hbm, idx_v, rows_v, sem):
    wid = lax.axis_index("s") * NC + lax.axis_index("c")
    base = wid * B_PER_W

    def step(i, _):
        off = base + i * CHUNK
        pltpu.sync_copy(idx_hbm.at[pl.ds(off, CHUNK)], idx_v)
        pltpu.async_copy(table_hbm.at[idx_v], rows_v, sem).wait()
        pltpu.sync_copy(rows_v, out_hbm.at[pl.ds(off, CHUNK)])
        return ()

    lax.fori_loop(0, N_CHUNKS, step, ())


@jax.jit
def _embed(indices_flat, table):
    mesh = plsc.VectorSubcoreMesh(core_axis_name="c", subcore_axis_name="s")
    f = pl.kernel(
        _gather_body,
        out_type=jax.ShapeDtypeStruct((TOTAL, D_MODEL), jnp.float32),
        mesh=mesh,
        scratch_types=[
            pltpu.VMEM((CHUNK,), jnp.int32),
            pltpu.VMEM((CHUNK, D_MODEL), jnp.float32),
            pltpu.SemaphoreType.DMA,
        ],
        compiler_params=pltpu.CompilerParams(use_tc_tiling_on_sc=False),
    )
    return f(indices_flat, table)


def kernel(indices, table):
    flat = indices.reshape(TOTAL).astype(jnp.int32)
    out = _embed(flat, table)
    return out.reshape(BATCH, SEQ, D_MODEL)


# speedup vs baseline: 1.0488x; 1.0488x over previous
import jax
import jax.numpy as jnp
from jax import lax
from jax.experimental import pallas as pl
from jax.experimental.pallas import tpu as pltpu
from jax.experimental.pallas import tpu_sc as plsc

NC = 2
NS = 16
NW = NC * NS

BATCH = 4096
SEQ = 200
D_MODEL = 64
TOTAL = BATCH * SEQ
B_PER_W = TOTAL // NW
CHUNK = 400
N_CHUNKS = B_PER_W // CHUNK
NBUF = 4
N_GROUPS = N_CHUNKS // NBUF


def _gather_body(idx_hbm, table_hbm, out_hbm, idx_v, rows_v, gsems, ssems):
    wid = lax.axis_index("s") * NC + lax.axis_index("c")
    base = wid * B_PER_W

    pltpu.sync_copy(idx_hbm.at[pl.ds(base, B_PER_W)], idx_v)

    def gather_copy(i, b):
        src = table_hbm.at[idx_v.at[pl.ds(i * CHUNK, CHUNK)]]
        return pltpu.make_async_copy(src, rows_v.at[b], gsems.at[b])

    def store_copy(i, b):
        dst = out_hbm.at[pl.ds(base + i * CHUNK, CHUNK)]
        return pltpu.make_async_copy(rows_v.at[b], dst, ssems.at[b])

    for b in range(NBUF):
        gather_copy(b, b).start()

    def group(g, _):
        for b in range(NBUF):
            i = g * NBUF + b
            gather_copy(i, b).wait()
            store_copy(i, b).start()
            store_copy(i, b).wait()
            gather_copy(i + NBUF, b).start()
        return ()

    lax.fori_loop(0, N_GROUPS - 1, group, ())

    for b in range(NBUF):
        i = (N_GROUPS - 1) * NBUF + b
        gather_copy(i, b).wait()
        store_copy(i, b).start()
    for b in range(NBUF):
        i = (N_GROUPS - 1) * NBUF + b
        store_copy(i, b).wait()


@jax.jit
def _embed(indices_flat, table):
    mesh = plsc.VectorSubcoreMesh(core_axis_name="c", subcore_axis_name="s")
    f = pl.kernel(
        _gather_body,
        out_type=jax.ShapeDtypeStruct((TOTAL, D_MODEL), jnp.float32),
        mesh=mesh,
        scratch_types=[
            pltpu.VMEM((B_PER_W,), jnp.int32),
            pltpu.VMEM((NBUF, CHUNK, D_MODEL), jnp.float32),
            pltpu.SemaphoreType.DMA((NBUF,)),
            pltpu.SemaphoreType.DMA((NBUF,)),
        ],
        compiler_params=pltpu.CompilerParams(use_tc_tiling_on_sc=False),
    )
    return f(indices_flat, table)


def kernel(indices, table):
    flat = indices.reshape(TOTAL).astype(jnp.int32)
    out = _embed(flat, table)
    return out.reshape(BATCH, SEQ, D_MODEL)


# speedup vs baseline: 1.2801x; 1.2205x over previous
import jax
import jax.numpy as jnp
from jax import lax
from jax.experimental import pallas as pl
from jax.experimental.pallas import tpu as pltpu
from jax.experimental.pallas import tpu_sc as plsc

NC = 2
NS = 16
NW = NC * NS

BATCH = 4096
SEQ = 200
D_MODEL = 64
D_PAD = 128
TOTAL = BATCH * SEQ
B_PER_W = TOTAL // NW
CHUNK = 200
N_CHUNKS = B_PER_W // CHUNK
NBUF = 4
N_GROUPS = N_CHUNKS // NBUF


def _gather_body(idx_hbm, table_hbm, out_hbm, idx_v, rows_v, gsems, ssems):
    wid = lax.axis_index("s") * NC + lax.axis_index("c")
    base = wid * B_PER_W

    pltpu.sync_copy(idx_hbm.at[pl.ds(base, B_PER_W)], idx_v)

    def gather_copy(i, b):
        src = table_hbm.at[idx_v.at[pl.ds(i * CHUNK, CHUNK)]]
        return pltpu.make_async_copy(src, rows_v.at[b], gsems.at[b])

    def store_copy(i, b):
        dst = out_hbm.at[pl.ds(base + i * CHUNK, CHUNK)]
        return pltpu.make_async_copy(rows_v.at[b], dst, ssems.at[b])

    for b in range(NBUF):
        gather_copy(b, b).start()

    def group(g, _):
        for b in range(NBUF):
            i = g * NBUF + b
            gather_copy(i, b).wait()
            store_copy(i, b).start()
            store_copy(i, b).wait()
            gather_copy(i + NBUF, b).start()
        return ()

    lax.fori_loop(0, N_GROUPS - 1, group, ())

    for b in range(NBUF):
        i = (N_GROUPS - 1) * NBUF + b
        gather_copy(i, b).wait()
        store_copy(i, b).start()
    for b in range(NBUF):
        i = (N_GROUPS - 1) * NBUF + b
        store_copy(i, b).wait()


@jax.jit
def _embed(indices_flat, table_padded):
    mesh = plsc.VectorSubcoreMesh(core_axis_name="c", subcore_axis_name="s")
    f = pl.kernel(
        _gather_body,
        out_type=jax.ShapeDtypeStruct((TOTAL, D_PAD), jnp.float32),
        mesh=mesh,
        scratch_types=[
            pltpu.VMEM((B_PER_W,), jnp.int32),
            pltpu.VMEM((NBUF, CHUNK, D_PAD), jnp.float32),
            pltpu.SemaphoreType.DMA((NBUF,)),
            pltpu.SemaphoreType.DMA((NBUF,)),
        ],
    )
    return f(indices_flat, table_padded)


def kernel(indices, table):
    flat = indices.reshape(TOTAL).astype(jnp.int32)
    table_padded = jnp.pad(table, ((0, 0), (0, D_PAD - D_MODEL)))
    out = _embed(flat, table_padded)
    return out[:, :D_MODEL].reshape(BATCH, SEQ, D_MODEL)
